# baseline (speedup 1.0000x reference)
import functools

import jax
import jax.numpy as jnp
from jax import lax
from jax.experimental import pallas as pl
from jax.experimental.pallas import tpu as pltpu
from jax.experimental.pallas import tpu_sc as plsc

_S = 64
_ROWS = _S + 1
_PAD = 72
_D = 128
_PER_W = 8
_NW = _S // _PER_W


def _sc_gather_body(node_hbm, ids_hbm, table_hbm, out_hbm,
                    idx_v, rows_v, nidx_v, nrow_v, sem):
    wid = lax.axis_index("s") * 2 + lax.axis_index("c")

    @pl.when(wid < _NW)
    def _():
        base = pl.multiple_of(wid * _PER_W, _PER_W)
        pltpu.sync_copy(ids_hbm.at[pl.ds(base, _PER_W)], idx_v)
        pltpu.async_copy(table_hbm.at[idx_v], rows_v, sem).wait()
        pltpu.sync_copy(rows_v, out_hbm.at[pl.ds(base, _PER_W)])

    @pl.when(wid == _NW)
    def _():
        pltpu.sync_copy(node_hbm, nidx_v)
        pltpu.async_copy(table_hbm.at[nidx_v], nrow_v, sem).wait()
        pltpu.sync_copy(nrow_v, out_hbm.at[pl.ds(_S, 1)])


@functools.partial(
    pl.kernel,
    out_type=jax.ShapeDtypeStruct((_PAD, _D), jnp.float32),
    mesh=plsc.VectorSubcoreMesh(core_axis_name="c", subcore_axis_name="s"),
    scratch_types=[
        pltpu.VMEM((_PER_W,), jnp.int32),
        pltpu.VMEM((_PER_W, _D), jnp.float32),
        pltpu.VMEM((1,), jnp.int32),
        pltpu.VMEM((1, _D), jnp.float32),
        pltpu.SemaphoreType.DMA,
    ],
)
def _sc_gather(node_hbm, ids_hbm, table_hbm, out_hbm,
               idx_v, rows_v, nidx_v, nrow_v, sem):
    _sc_gather_body(node_hbm, ids_hbm, table_hbm, out_hbm,
                    idx_v, rows_v, nidx_v, nrow_v, sem)


def _tc_dense_body(rows_ref, wq, bq, wk, bk, wv, bv, out_ref):
    row_id2 = lax.broadcasted_iota(jnp.int32, (_PAD, _D), 0)
    r = jnp.where(row_id2 < _ROWS, rows_ref[...], 0.0)
    self_row = r[_S:_S + 1]
    q = jnp.dot(self_row, wq[...],
                preferred_element_type=jnp.float32) + bq[...]
    k = jnp.dot(r, wk[...],
                preferred_element_type=jnp.float32) + bk[...]
    v = jnp.dot(r, wv[...],
                preferred_element_type=jnp.float32) + bv[...]

    s = jnp.dot(k, q.T, preferred_element_type=jnp.float32)
    row_id = lax.broadcasted_iota(jnp.int32, (_PAD, 1), 0)
    s = jnp.where(row_id < _ROWS, s, -jnp.inf)
    m = jnp.max(s)
    e = jnp.exp(s - m)
    p = e / jnp.sum(e)
    mix = jnp.sum(p * v, axis=0, keepdims=True)

    f = jnp.tanh(mix)
    norm = jnp.maximum(jnp.sqrt(jnp.sum(f * f)), 1e-12)
    out_ref[...] = f / norm


def kernel(table, Wq, bq, Wk, bk, Wv, bv, node, neigh_ids):
    node1 = jnp.reshape(node, (1,)).astype(jnp.int32)
    gathered = _sc_gather(node1, neigh_ids, table)
    return pl.pallas_call(
        _tc_dense_body,
        out_shape=jax.ShapeDtypeStruct((1, _D), jnp.float32),
        in_specs=[
            pl.BlockSpec(memory_space=pltpu.VMEM),
            pl.BlockSpec(memory_space=pltpu.VMEM),
            pl.BlockSpec(memory_space=pltpu.VMEM),
            pl.BlockSpec(memory_space=pltpu.VMEM),
            pl.BlockSpec(memory_space=pltpu.VMEM),
            pl.BlockSpec(memory_space=pltpu.VMEM),
            pl.BlockSpec(memory_space=pltpu.VMEM),
        ],
        out_specs=pl.BlockSpec(memory_space=pltpu.VMEM),
    )(gathered,
      Wq, jnp.reshape(bq, (1, _D)),
      Wk, jnp.reshape(bk, (1, _D)),
      Wv, jnp.reshape(bv, (1, _D)))


# speedup vs baseline: 1.1418x; 1.1418x over previous
import functools

import jax
import jax.numpy as jnp
from jax import lax
from jax.experimental import pallas as pl
from jax.experimental.pallas import tpu as pltpu
from jax.experimental.pallas import tpu_sc as plsc

_S = 64
_ROWS = _S + 1
_PAD = 72
_D = 128
_PER_W = 8
_NW = _S // _PER_W


def _sc_gather_body(node_hbm, ids_hbm, table_hbm, out_hbm,
                    idx_v, rows_v, nidx_v, nrow_v, sem):
    wid = lax.axis_index("s")

    @pl.when(wid < _NW)
    def _():
        base = pl.multiple_of(wid * _PER_W, _PER_W)
        pltpu.sync_copy(ids_hbm.at[pl.ds(base, _PER_W)], idx_v)
        pltpu.async_copy(table_hbm.at[idx_v], rows_v, sem).wait()
        pltpu.sync_copy(rows_v, out_hbm.at[pl.ds(base, _PER_W)])

    @pl.when(wid == _NW)
    def _():
        pltpu.sync_copy(node_hbm, nidx_v)
        pltpu.async_copy(table_hbm.at[nidx_v], nrow_v, sem).wait()
        pltpu.sync_copy(nrow_v, out_hbm.at[pl.ds(_S, 1)])


@functools.partial(
    pl.kernel,
    out_type=jax.ShapeDtypeStruct((_PAD, _D), jnp.float32),
    mesh=plsc.VectorSubcoreMesh(core_axis_name="c", subcore_axis_name="s",
                                num_cores=1),
    scratch_types=[
        pltpu.VMEM((_PER_W,), jnp.int32),
        pltpu.VMEM((_PER_W, _D), jnp.float32),
        pltpu.VMEM((1,), jnp.int32),
        pltpu.VMEM((1, _D), jnp.float32),
        pltpu.SemaphoreType.DMA,
    ],
)
def _sc_gather(node_hbm, ids_hbm, table_hbm, out_hbm,
               idx_v, rows_v, nidx_v, nrow_v, sem):
    _sc_gather_body(node_hbm, ids_hbm, table_hbm, out_hbm,
                    idx_v, rows_v, nidx_v, nrow_v, sem)


def _tc_dense_body(rows_ref, wq, bq, wk, bk, wv, bv, out_ref):
    row_id2 = lax.broadcasted_iota(jnp.int32, (_PAD, _D), 0)
    r = jnp.where(row_id2 < _ROWS, rows_ref[...], 0.0)
    self_row = r[_S:_S + 1]
    q = jnp.dot(self_row, wq[...],
                preferred_element_type=jnp.float32) + bq[...]
    k = jnp.dot(r, wk[...],
                preferred_element_type=jnp.float32) + bk[...]
    v = jnp.dot(r, wv[...],
                preferred_element_type=jnp.float32) + bv[...]

    s = jnp.dot(k, q.T, preferred_element_type=jnp.float32)
    row_id = lax.broadcasted_iota(jnp.int32, (_PAD, 1), 0)
    s = jnp.where(row_id < _ROWS, s, -jnp.inf)
    m = jnp.max(s)
    e = jnp.exp(s - m)
    p = e / jnp.sum(e)
    mix = jnp.sum(p * v, axis=0, keepdims=True)

    f = jnp.tanh(mix)
    norm = jnp.maximum(jnp.sqrt(jnp.sum(f * f)), 1e-12)
    out_ref[...] = f / norm


def kernel(table, Wq, bq, Wk, bk, Wv, bv, node, neigh_ids):
    node1 = jnp.reshape(node, (1,)).astype(jnp.int32)
    gathered = _sc_gather(node1, neigh_ids, table)
    return pl.pallas_call(
        _tc_dense_body,
        out_shape=jax.ShapeDtypeStruct((1, _D), jnp.float32),
        in_specs=[
            pl.BlockSpec(memory_space=pltpu.VMEM),
            pl.BlockSpec(memory_space=pltpu.VMEM),
            pl.BlockSpec(memory_space=pltpu.VMEM),
            pl.BlockSpec(memory_space=pltpu.VMEM),
            pl.BlockSpec(memory_space=pltpu.VMEM),
            pl.BlockSpec(memory_space=pltpu.VMEM),
            pl.BlockSpec(memory_space=pltpu.VMEM),
        ],
        out_specs=pl.BlockSpec(memory_space=pltpu.VMEM),
    )(gathered,
      Wq, jnp.reshape(bq, (1, _D)),
      Wk, jnp.reshape(bk, (1, _D)),
      Wv, jnp.reshape(bv, (1, _D)))
